# Initial kernel scaffold; baseline (speedup 1.0000x reference)
import jax
import jax.numpy as jnp
from jax.experimental import pallas as pl


def kernel(x, edge_index, W11, b11, W12, b12, g1, be1, W21, b21, W22, b22, g2, be2, W31, b31, W32, b32, g3, be3, Wf, bf):
    raise NotImplementedError("write your pallas kernel here")


# speedup vs baseline: 14.4286x; 14.4286x over previous
import functools

import jax
import jax.numpy as jnp
from jax import lax
from jax.experimental import pallas as pl
from jax.experimental.pallas import tpu as pltpu
from jax.experimental.pallas import tpu_sc as plsc

N_NODES = 100000
N_EDGES = 6400000
HID = 10
OUT_DIM = 4

NC = 2
NS = 16
NW = NC * NS
CH = 128
CPT = 1563
EPT = CH * CPT
E_PAD = EPT * NW
N_PAD = 100096
RPT = N_PAD // NS
ZCH = 272
DP = 16

_f32 = jnp.float32


def _make_segsum():
  mesh = plsc.VectorSubcoreMesh(core_axis_name="c", subcore_axis_name="s", num_cores=NC, num_subcores=NS)

  @functools.partial(
      pl.kernel,
      mesh=mesh,
      compiler_params=pltpu.CompilerParams(use_tc_tiling_on_sc=False),
      out_type=jax.ShapeDtypeStruct((2 * N_PAD, DP), _f32),
      scratch_types=[
          pltpu.VMEM((CH,), jnp.int32),
          pltpu.VMEM((CH,), jnp.int32),
          pltpu.VMEM((CH, DP), _f32),
          pltpu.VMEM((ZCH, DP), _f32),
          pltpu.VMEM_SHARED((N_PAD, DP), _f32),
      ],
  )
  def seg(table, srcp, dstp, zrows, out, src_v, dst_v, rows_v, zb_v, acc):
    c = lax.axis_index("c")
    s = lax.axis_index("s")
    w = c * NS + s
    pltpu.sync_copy(zrows, zb_v)
    zbase = s * RPT
    for z in range(RPT // ZCH):
      pltpu.sync_copy(zb_v, acc.at[pl.ds(zbase + z * ZCH, ZCH)])
    plsc.subcore_barrier()

    def step(t, carry):
      base = w * EPT + t * CH
      pltpu.sync_copy(srcp.at[pl.ds(base, CH)], src_v)
      pltpu.sync_copy(dstp.at[pl.ds(base, CH)], dst_v)
      pltpu.sync_copy(table.at[src_v], rows_v)
      pltpu.sync_copy(rows_v, acc.at[dst_v], add=True)
      return carry

    lax.fori_loop(0, CPT, step, 0)
    plsc.subcore_barrier()
    obase = c * N_PAD + zbase
    for z in range(RPT // ZCH):
      pltpu.sync_copy(acc.at[pl.ds(zbase + z * ZCH, ZCH)], zb_v)
      pltpu.sync_copy(zb_v, out.at[pl.ds(obase + z * ZCH, ZCH)])

  return seg


_sc_cache = {}


def _segsum():
  if "seg" not in _sc_cache:
    _sc_cache["seg"] = _make_segsum()
  return _sc_cache["seg"]


def _make_degree():
  mesh = plsc.VectorSubcoreMesh(core_axis_name="c", subcore_axis_name="s", num_cores=NC, num_subcores=NS)

  @functools.partial(
      pl.kernel,
      mesh=mesh,
      compiler_params=pltpu.CompilerParams(use_tc_tiling_on_sc=False),
      out_type=jax.ShapeDtypeStruct((2 * N_PAD, DP), _f32),
      scratch_types=[
          pltpu.VMEM((CH,), jnp.int32),
          pltpu.VMEM((CH, DP), _f32),
          pltpu.VMEM((ZCH, DP), _f32),
          pltpu.VMEM_SHARED((N_PAD, DP), _f32),
      ],
  )
  def deg(idxp, ones_h, zrows, out, idx_v, ones_v, zb_v, acc):
    c = lax.axis_index("c")
    s = lax.axis_index("s")
    w = c * NS + s
    pltpu.sync_copy(ones_h, ones_v)
    pltpu.sync_copy(zrows, zb_v)
    zbase = s * RPT
    for z in range(RPT // ZCH):
      pltpu.sync_copy(zb_v, acc.at[pl.ds(zbase + z * ZCH, ZCH)])
    plsc.subcore_barrier()

    def step(t, carry):
      base = w * EPT + t * CH
      pltpu.sync_copy(idxp.at[pl.ds(base, CH)], idx_v)
      pltpu.sync_copy(ones_v, acc.at[idx_v], add=True)
      return carry

    lax.fori_loop(0, CPT, step, 0)
    plsc.subcore_barrier()
    obase = c * N_PAD + zbase
    for z in range(RPT // ZCH):
      pltpu.sync_copy(acc.at[pl.ds(zbase + z * ZCH, ZCH)], zb_v)
      pltpu.sync_copy(zb_v, out.at[pl.ds(obase + z * ZCH, ZCH)])

  return deg


def _degree_k():
  if "deg" not in _sc_cache:
    _sc_cache["deg"] = _make_degree()
  return _sc_cache["deg"]


BN = 10000
NB = N_NODES // BN


def _gin_mlp_a(din):
  def body(x_ref, a0_ref, a1_ref, w1_ref, b1_ref, w2_ref, b2_ref,
           h2_ref, st_ref, scr):
    i = pl.program_id(0)
    h = x_ref[...] + a0_ref[...] + a1_ref[...]
    if din == 1:
      h1 = h * w1_ref[...] + b1_ref[...]
    else:
      h1 = jnp.dot(h, w1_ref[...], preferred_element_type=_f32) + b1_ref[...]
    h2 = jnp.dot(h1, w2_ref[...], preferred_element_type=_f32) + b2_ref[...]
    h2_ref[...] = h2
    blk = jnp.stack([jnp.sum(h2, axis=0), jnp.sum(h2 * h2, axis=0)])

    @pl.when(i == 0)
    def _():
      scr[...] = blk

    @pl.when(i > 0)
    def _():
      scr[...] = scr[...] + blk

    st_ref[...] = scr[...]

  return pl.pallas_call(
      body,
      grid=(NB,),
      in_specs=[
          pl.BlockSpec((BN, din), lambda i: (i, 0)),
          pl.BlockSpec((BN, din), lambda i: (i, 0)),
          pl.BlockSpec((BN, din), lambda i: (i, 0)),
          pl.BlockSpec((din, HID), lambda i: (0, 0)),
          pl.BlockSpec((1, HID), lambda i: (0, 0)),
          pl.BlockSpec((HID, HID), lambda i: (0, 0)),
          pl.BlockSpec((1, HID), lambda i: (0, 0)),
      ],
      out_specs=[
          pl.BlockSpec((BN, HID), lambda i: (i, 0)),
          pl.BlockSpec((2, HID), lambda i: (0, 0)),
      ],
      out_shape=[
          jax.ShapeDtypeStruct((N_NODES, HID), _f32),
          jax.ShapeDtypeStruct((2, HID), _f32),
      ],
      scratch_shapes=[pltpu.VMEM((2, HID), _f32)],
  )


_gin_mlp_a1 = _gin_mlp_a(1)
_gin_mlp_a10 = _gin_mlp_a(HID)


def _bn_body(h2_ref, st_ref, g_ref, be_ref, o_ref):
  inv_n = 1.0 / N_NODES
  mu = st_ref[0:1, :] * inv_n
  var = st_ref[1:2, :] * inv_n - mu * mu
  scale = lax.rsqrt(var + 1e-5) * g_ref[...]
  o_ref[...] = (h2_ref[...] - mu) * scale + be_ref[...]


_bn_norm = pl.pallas_call(
    _bn_body,
    grid=(NB,),
    in_specs=[
        pl.BlockSpec((BN, HID), lambda i: (i, 0)),
        pl.BlockSpec((2, HID), lambda i: (0, 0)),
        pl.BlockSpec((1, HID), lambda i: (0, 0)),
        pl.BlockSpec((1, HID), lambda i: (0, 0)),
    ],
    out_specs=pl.BlockSpec((BN, HID), lambda i: (i, 0)),
    out_shape=jax.ShapeDtypeStruct((N_NODES, HID), _f32),
)


def _readout_body(x1_ref, x2_ref, x3_ref, w1_ref, w2_ref, w3_ref, bf_ref,
                  s_ref):
  logits = (
      jnp.dot(x1_ref[...], w1_ref[...], preferred_element_type=_f32)
      + jnp.dot(x2_ref[...], w2_ref[...], preferred_element_type=_f32)
      + jnp.dot(x3_ref[...], w3_ref[...], preferred_element_type=_f32)
      + bf_ref[...])
  m = jnp.max(logits, axis=-1, keepdims=True)
  e = jnp.exp(logits - m)
  s_ref[...] = e / jnp.sum(e, axis=-1, keepdims=True)


_readout = pl.pallas_call(
    _readout_body,
    grid=(NB,),
    in_specs=[
        pl.BlockSpec((BN, HID), lambda i: (i, 0)),
        pl.BlockSpec((BN, HID), lambda i: (i, 0)),
        pl.BlockSpec((BN, HID), lambda i: (i, 0)),
        pl.BlockSpec((HID, OUT_DIM), lambda i: (0, 0)),
        pl.BlockSpec((HID, OUT_DIM), lambda i: (0, 0)),
        pl.BlockSpec((HID, OUT_DIM), lambda i: (0, 0)),
        pl.BlockSpec((1, OUT_DIM), lambda i: (0, 0)),
    ],
    out_specs=pl.BlockSpec((BN, OUT_DIM), lambda i: (i, 0)),
    out_shape=jax.ShapeDtypeStruct((N_NODES, OUT_DIM), _f32),
)

_TWO_M = float(N_EDGES)


BN_M = 4000
NB_M = N_NODES // BN_M


def _mod_body(s_ref, p0_ref, p1_ref, d0_ref, d1_ref, q_ref, scr):
  i = pl.program_id(0)
  sv = s_ref[...]
  agg = p0_ref[...] + p1_ref[...]
  deg = d0_ref[...] + d1_ref[...]
  pos_blk = jnp.sum(sv * agg)

  @pl.when(i == 0)
  def _():
    scr[0] = pos_blk
    for j in range(OUT_DIM):
      scr[1 + j] = jnp.sum(deg * sv[:, j:j + 1])

  @pl.when(i > 0)
  def _():
    scr[0] = scr[0] + pos_blk
    for j in range(OUT_DIM):
      scr[1 + j] = scr[1 + j] + jnp.sum(deg * sv[:, j:j + 1])

  neg = (scr[1] * scr[1] + scr[2] * scr[2] + scr[3] * scr[3]
         + scr[4] * scr[4]) / _TWO_M
  q_ref[...] = jnp.full((1, 1), (scr[0] - neg) / _TWO_M)


_modularity = pl.pallas_call(
    _mod_body,
    grid=(NB_M,),
    in_specs=[
        pl.BlockSpec((BN_M, OUT_DIM), lambda i: (i, 0)),
        pl.BlockSpec((BN_M, OUT_DIM), lambda i: (i, 0)),
        pl.BlockSpec((BN_M, OUT_DIM), lambda i: (i, 0)),
        pl.BlockSpec((BN_M, 1), lambda i: (i, 0)),
        pl.BlockSpec((BN_M, 1), lambda i: (i, 0)),
    ],
    out_specs=pl.BlockSpec((1, 1), lambda i: (0, 0)),
    out_shape=jax.ShapeDtypeStruct((1, 1), _f32),
    scratch_shapes=[pltpu.SMEM((8,), _f32)],
)


def _gin_layer(xin, part, w1, b1, w2, b2, g, be, din):
  mlp = _gin_mlp_a1 if din == 1 else _gin_mlp_a10
  h2, st = mlp(xin, part[:N_NODES], part[N_PAD:N_PAD + N_NODES],
               w1.reshape(din, HID), b1.reshape(1, HID),
               w2, b2.reshape(1, HID))
  return _bn_norm(h2, st, g.reshape(1, HID), be.reshape(1, HID))


def kernel(x, edge_index, W11, b11, W12, b12, g1, be1, W21, b21, W22, b22,
           g2, be2, W31, b31, W32, b32, g3, be3, Wf, bf):
  src = edge_index[0]
  dst = edge_index[1]
  padv = jnp.full((E_PAD - N_EDGES,), N_NODES, jnp.int32)
  srcp = jnp.concatenate([src, padv])
  dstp = jnp.concatenate([dst, padv])
  zr = jnp.zeros((ZCH, DP), _f32)
  ones_h = jnp.ones((CH, DP), _f32)

  def table(a):
    return jnp.pad(a, ((0, 1), (0, DP - a.shape[1])))

  dg = _degree_k()(srcp, ones_h, zr)

  p = _segsum()(table(x), srcp, dstp, zr)
  x11 = _gin_layer(x, p[:, :1], W11, b11, W12, b12, g1, be1, 1)

  p = _segsum()(table(x11), srcp, dstp, zr)
  x12 = _gin_layer(x11, p[:, :HID], W21, b21, W22, b22, g2, be2, HID)

  p = _segsum()(table(x12), srcp, dstp, zr)
  x13 = _gin_layer(x12, p[:, :HID], W31, b31, W32, b32, g3, be3, HID)

  s = _readout(x11, x12, x13, Wf[:HID], Wf[HID:2 * HID], Wf[2 * HID:],
               bf.reshape(1, OUT_DIM))

  sp = _segsum()(table(s), srcp, dstp, zr)
  q = _modularity(s, sp[:N_NODES, :OUT_DIM], sp[N_PAD:N_PAD + N_NODES, :OUT_DIM],
                  dg[:N_NODES, :1], dg[N_PAD:N_PAD + N_NODES, :1])
  return s, q[0, 0]


# speedup vs baseline: 48.2137x; 3.3415x over previous
import functools

import jax
import jax.numpy as jnp
from jax import lax
from jax.experimental import pallas as pl
from jax.experimental.pallas import tpu as pltpu
from jax.experimental.pallas import tpu_sc as plsc

N_NODES = 100000
N_EDGES = 6400000
HID = 10
OUT_DIM = 4

NC = 2
NS = 16
NW = NC * NS
CH = 128
KB = 8
CPT = 1568
EPT = CH * CPT
E_PAD = EPT * NW
NBATCH = CPT // KB
N_PAD = 100096
RPT = N_PAD // NS
ZCH = 272
DP = 16

_f32 = jnp.float32


def _make_segsum():
  mesh = plsc.VectorSubcoreMesh(core_axis_name="c", subcore_axis_name="s", num_cores=NC, num_subcores=NS)

  @functools.partial(
      pl.kernel,
      mesh=mesh,
      compiler_params=pltpu.CompilerParams(use_tc_tiling_on_sc=False),
      out_type=jax.ShapeDtypeStruct((2 * N_PAD, DP), _f32),
      scratch_types=[
          pltpu.VMEM((KB, CH), jnp.int32),
          pltpu.VMEM((KB, CH), jnp.int32),
          pltpu.VMEM((KB, CH, DP), _f32),
          pltpu.VMEM((ZCH, DP), _f32),
          pltpu.VMEM_SHARED((N_PAD, DP), _f32),
          pltpu.SemaphoreType.DMA,
          pltpu.SemaphoreType.DMA,
          pltpu.SemaphoreType.DMA,
      ],
  )
  def seg(table, srcp, dstp, zrows, out, sidx_v, didx_v, rows_v, zb_v, acc,
          isem, gsem, ssem):
    c = lax.axis_index("c")
    s = lax.axis_index("s")
    w = c * NS + s
    pltpu.sync_copy(zrows, zb_v)
    zbase = s * RPT
    for z in range(RPT // ZCH):
      pltpu.sync_copy(zb_v, acc.at[pl.ds(zbase + z * ZCH, ZCH)])
    plsc.subcore_barrier()

    def step(b, carry):
      rbase = w * CPT + b * KB
      i1 = pltpu.async_copy(srcp.at[pl.ds(rbase, KB)], sidx_v, isem)
      i2 = pltpu.async_copy(dstp.at[pl.ds(rbase, KB)], didx_v, isem)
      i1.wait()
      i2.wait()
      gs = [pltpu.async_copy(table.at[sidx_v.at[j]], rows_v.at[j], gsem)
            for j in range(KB)]
      ss = []
      for j in range(KB):
        gs[j].wait()
        ss.append(pltpu.async_copy(rows_v.at[j], acc.at[didx_v.at[j]], ssem,
                                   add=True))
      for h in ss:
        h.wait()
      return carry

    lax.fori_loop(0, NBATCH, step, 0)
    plsc.subcore_barrier()
    obase = c * N_PAD + zbase
    for z in range(RPT // ZCH):
      pltpu.sync_copy(acc.at[pl.ds(zbase + z * ZCH, ZCH)], zb_v)
      pltpu.sync_copy(zb_v, out.at[pl.ds(obase + z * ZCH, ZCH)])

  return seg


_sc_cache = {}


def _segsum():
  if "seg" not in _sc_cache:
    _sc_cache["seg"] = _make_segsum()
  return _sc_cache["seg"]


def _make_degree():
  mesh = plsc.VectorSubcoreMesh(core_axis_name="c", subcore_axis_name="s", num_cores=NC, num_subcores=NS)

  @functools.partial(
      pl.kernel,
      mesh=mesh,
      compiler_params=pltpu.CompilerParams(use_tc_tiling_on_sc=False),
      out_type=jax.ShapeDtypeStruct((2 * N_PAD, DP), _f32),
      scratch_types=[
          pltpu.VMEM((KB, CH), jnp.int32),
          pltpu.VMEM((CH, DP), _f32),
          pltpu.VMEM((ZCH, DP), _f32),
          pltpu.VMEM_SHARED((N_PAD, DP), _f32),
          pltpu.SemaphoreType.DMA,
          pltpu.SemaphoreType.DMA,
      ],
  )
  def deg(idxp, ones_h, zrows, out, didx_v, ones_v, zb_v, acc, isem, ssem):
    c = lax.axis_index("c")
    s = lax.axis_index("s")
    w = c * NS + s
    pltpu.sync_copy(ones_h, ones_v)
    pltpu.sync_copy(zrows, zb_v)
    zbase = s * RPT
    for z in range(RPT // ZCH):
      pltpu.sync_copy(zb_v, acc.at[pl.ds(zbase + z * ZCH, ZCH)])
    plsc.subcore_barrier()

    def step(b, carry):
      rbase = w * CPT + b * KB
      pltpu.async_copy(idxp.at[pl.ds(rbase, KB)], didx_v, isem).wait()
      ss = [pltpu.async_copy(ones_v, acc.at[didx_v.at[j]], ssem, add=True)
            for j in range(KB)]
      for h in ss:
        h.wait()
      return carry

    lax.fori_loop(0, NBATCH, step, 0)
    plsc.subcore_barrier()
    obase = c * N_PAD + zbase
    for z in range(RPT // ZCH):
      pltpu.sync_copy(acc.at[pl.ds(zbase + z * ZCH, ZCH)], zb_v)
      pltpu.sync_copy(zb_v, out.at[pl.ds(obase + z * ZCH, ZCH)])

  return deg


def _degree_k():
  if "deg" not in _sc_cache:
    _sc_cache["deg"] = _make_degree()
  return _sc_cache["deg"]


BN = 10000
NB = N_NODES // BN


def _gin_mlp_a(din):
  def body(x_ref, a0_ref, a1_ref, w1_ref, b1_ref, w2_ref, b2_ref,
           h2_ref, st_ref, scr):
    i = pl.program_id(0)
    h = x_ref[...] + a0_ref[...] + a1_ref[...]
    if din == 1:
      h1 = h * w1_ref[...] + b1_ref[...]
    else:
      h1 = jnp.dot(h, w1_ref[...], preferred_element_type=_f32) + b1_ref[...]
    h2 = jnp.dot(h1, w2_ref[...], preferred_element_type=_f32) + b2_ref[...]
    h2_ref[...] = h2
    blk = jnp.stack([jnp.sum(h2, axis=0), jnp.sum(h2 * h2, axis=0)])

    @pl.when(i == 0)
    def _():
      scr[...] = blk

    @pl.when(i > 0)
    def _():
      scr[...] = scr[...] + blk

    st_ref[...] = scr[...]

  return pl.pallas_call(
      body,
      grid=(NB,),
      in_specs=[
          pl.BlockSpec((BN, din), lambda i: (i, 0)),
          pl.BlockSpec((BN, din), lambda i: (i, 0)),
          pl.BlockSpec((BN, din), lambda i: (i, 0)),
          pl.BlockSpec((din, HID), lambda i: (0, 0)),
          pl.BlockSpec((1, HID), lambda i: (0, 0)),
          pl.BlockSpec((HID, HID), lambda i: (0, 0)),
          pl.BlockSpec((1, HID), lambda i: (0, 0)),
      ],
      out_specs=[
          pl.BlockSpec((BN, HID), lambda i: (i, 0)),
          pl.BlockSpec((2, HID), lambda i: (0, 0)),
      ],
      out_shape=[
          jax.ShapeDtypeStruct((N_NODES, HID), _f32),
          jax.ShapeDtypeStruct((2, HID), _f32),
      ],
      scratch_shapes=[pltpu.VMEM((2, HID), _f32)],
  )


_gin_mlp_a1 = _gin_mlp_a(1)
_gin_mlp_a10 = _gin_mlp_a(HID)


def _bn_body(h2_ref, st_ref, g_ref, be_ref, o_ref):
  inv_n = 1.0 / N_NODES
  mu = st_ref[0:1, :] * inv_n
  var = st_ref[1:2, :] * inv_n - mu * mu
  scale = lax.rsqrt(var + 1e-5) * g_ref[...]
  o_ref[...] = (h2_ref[...] - mu) * scale + be_ref[...]


_bn_norm = pl.pallas_call(
    _bn_body,
    grid=(NB,),
    in_specs=[
        pl.BlockSpec((BN, HID), lambda i: (i, 0)),
        pl.BlockSpec((2, HID), lambda i: (0, 0)),
        pl.BlockSpec((1, HID), lambda i: (0, 0)),
        pl.BlockSpec((1, HID), lambda i: (0, 0)),
    ],
    out_specs=pl.BlockSpec((BN, HID), lambda i: (i, 0)),
    out_shape=jax.ShapeDtypeStruct((N_NODES, HID), _f32),
)


def _readout_body(x1_ref, x2_ref, x3_ref, w1_ref, w2_ref, w3_ref, bf_ref,
                  s_ref):
  logits = (
      jnp.dot(x1_ref[...], w1_ref[...], preferred_element_type=_f32)
      + jnp.dot(x2_ref[...], w2_ref[...], preferred_element_type=_f32)
      + jnp.dot(x3_ref[...], w3_ref[...], preferred_element_type=_f32)
      + bf_ref[...])
  m = jnp.max(logits, axis=-1, keepdims=True)
  e = jnp.exp(logits - m)
  s_ref[...] = e / jnp.sum(e, axis=-1, keepdims=True)


_readout = pl.pallas_call(
    _readout_body,
    grid=(NB,),
    in_specs=[
        pl.BlockSpec((BN, HID), lambda i: (i, 0)),
        pl.BlockSpec((BN, HID), lambda i: (i, 0)),
        pl.BlockSpec((BN, HID), lambda i: (i, 0)),
        pl.BlockSpec((HID, OUT_DIM), lambda i: (0, 0)),
        pl.BlockSpec((HID, OUT_DIM), lambda i: (0, 0)),
        pl.BlockSpec((HID, OUT_DIM), lambda i: (0, 0)),
        pl.BlockSpec((1, OUT_DIM), lambda i: (0, 0)),
    ],
    out_specs=pl.BlockSpec((BN, OUT_DIM), lambda i: (i, 0)),
    out_shape=jax.ShapeDtypeStruct((N_NODES, OUT_DIM), _f32),
)

_TWO_M = float(N_EDGES)


BN_M = 4000
NB_M = N_NODES // BN_M


def _mod_body(s_ref, p0_ref, p1_ref, d0_ref, d1_ref, q_ref, scr):
  i = pl.program_id(0)
  sv = s_ref[...]
  agg = p0_ref[...] + p1_ref[...]
  deg = d0_ref[...] + d1_ref[...]
  pos_blk = jnp.sum(sv * agg)

  @pl.when(i == 0)
  def _():
    scr[0] = pos_blk
    for j in range(OUT_DIM):
      scr[1 + j] = jnp.sum(deg * sv[:, j:j + 1])

  @pl.when(i > 0)
  def _():
    scr[0] = scr[0] + pos_blk
    for j in range(OUT_DIM):
      scr[1 + j] = scr[1 + j] + jnp.sum(deg * sv[:, j:j + 1])

  neg = (scr[1] * scr[1] + scr[2] * scr[2] + scr[3] * scr[3]
         + scr[4] * scr[4]) / _TWO_M
  q_ref[...] = jnp.full((1, 1), (scr[0] - neg) / _TWO_M)


_modularity = pl.pallas_call(
    _mod_body,
    grid=(NB_M,),
    in_specs=[
        pl.BlockSpec((BN_M, OUT_DIM), lambda i: (i, 0)),
        pl.BlockSpec((BN_M, OUT_DIM), lambda i: (i, 0)),
        pl.BlockSpec((BN_M, OUT_DIM), lambda i: (i, 0)),
        pl.BlockSpec((BN_M, 1), lambda i: (i, 0)),
        pl.BlockSpec((BN_M, 1), lambda i: (i, 0)),
    ],
    out_specs=pl.BlockSpec((1, 1), lambda i: (0, 0)),
    out_shape=jax.ShapeDtypeStruct((1, 1), _f32),
    scratch_shapes=[pltpu.SMEM((8,), _f32)],
)


def _gin_layer(xin, part, w1, b1, w2, b2, g, be, din):
  mlp = _gin_mlp_a1 if din == 1 else _gin_mlp_a10
  h2, st = mlp(xin, part[:N_NODES], part[N_PAD:N_PAD + N_NODES],
               w1.reshape(din, HID), b1.reshape(1, HID),
               w2, b2.reshape(1, HID))
  return _bn_norm(h2, st, g.reshape(1, HID), be.reshape(1, HID))


def kernel(x, edge_index, W11, b11, W12, b12, g1, be1, W21, b21, W22, b22,
           g2, be2, W31, b31, W32, b32, g3, be3, Wf, bf):
  src = edge_index[0]
  dst = edge_index[1]
  padv = jnp.full((E_PAD - N_EDGES,), N_NODES, jnp.int32)
  srcp = jnp.concatenate([src, padv]).reshape(-1, CH)
  dstp = jnp.concatenate([dst, padv]).reshape(-1, CH)
  zr = jnp.zeros((ZCH, DP), _f32)
  ones_h = jnp.ones((CH, DP), _f32)

  def table(a):
    return jnp.pad(a, ((0, 1), (0, DP - a.shape[1])))

  dg = _degree_k()(srcp, ones_h, zr)

  p = _segsum()(table(x), srcp, dstp, zr)
  x11 = _gin_layer(x, p[:, :1], W11, b11, W12, b12, g1, be1, 1)

  p = _segsum()(table(x11), srcp, dstp, zr)
  x12 = _gin_layer(x11, p[:, :HID], W21, b21, W22, b22, g2, be2, HID)

  p = _segsum()(table(x12), srcp, dstp, zr)
  x13 = _gin_layer(x12, p[:, :HID], W31, b31, W32, b32, g3, be3, HID)

  s = _readout(x11, x12, x13, Wf[:HID], Wf[HID:2 * HID], Wf[2 * HID:],
               bf.reshape(1, OUT_DIM))

  sp = _segsum()(table(s), srcp, dstp, zr)
  q = _modularity(s, sp[:N_NODES, :OUT_DIM], sp[N_PAD:N_PAD + N_NODES, :OUT_DIM],
                  dg[:N_NODES, :1], dg[N_PAD:N_PAD + N_NODES, :1])
  return s, q[0, 0]


# speedup vs baseline: 50.2450x; 1.0421x over previous
import functools

import jax
import jax.numpy as jnp
from jax import lax
from jax.experimental import pallas as pl
from jax.experimental.pallas import tpu as pltpu
from jax.experimental.pallas import tpu_sc as plsc

N_NODES = 100000
N_EDGES = 6400000
HID = 10
OUT_DIM = 4

NC = 2
NS = 16
NW = NC * NS
CH = 128
KB = 8
CPT = 1568
EPT = CH * CPT
E_PAD = EPT * NW
NBATCH = CPT // KB
N_PAD = 100096
RPT = N_PAD // NS
ZCH = 272
DP = 16

_f32 = jnp.float32


def _make_segsum():
  mesh = plsc.VectorSubcoreMesh(core_axis_name="c", subcore_axis_name="s", num_cores=NC, num_subcores=NS)

  @functools.partial(
      pl.kernel,
      mesh=mesh,
      compiler_params=pltpu.CompilerParams(use_tc_tiling_on_sc=False),
      out_type=jax.ShapeDtypeStruct((2 * N_PAD, DP), _f32),
      scratch_types=[
          pltpu.VMEM((KB, CH), jnp.int32),
          pltpu.VMEM((KB, CH), jnp.int32),
          pltpu.VMEM((KB, CH, DP), _f32),
          pltpu.VMEM((ZCH, DP), _f32),
          pltpu.VMEM_SHARED((N_PAD, DP), _f32),
          pltpu.SemaphoreType.DMA,
          pltpu.SemaphoreType.DMA,
          pltpu.SemaphoreType.DMA,
      ],
  )
  def seg(table, srcp, dstp, zrows, out, sidx_v, didx_v, rows_v, zb_v, acc,
          isem, gsem, ssem):
    c = lax.axis_index("c")
    s = lax.axis_index("s")
    w = c * NS + s
    pltpu.sync_copy(zrows, zb_v)
    zbase = s * RPT
    for z in range(RPT // ZCH):
      pltpu.sync_copy(zb_v, acc.at[pl.ds(zbase + z * ZCH, ZCH)])
    plsc.subcore_barrier()

    def step(b, carry):
      rbase = w * CPT + b * KB
      i1 = pltpu.async_copy(srcp.at[pl.ds(rbase, KB)], sidx_v, isem)
      i2 = pltpu.async_copy(dstp.at[pl.ds(rbase, KB)], didx_v, isem)
      i1.wait()
      i2.wait()
      gs = [pltpu.async_copy(table.at[sidx_v.at[j]], rows_v.at[j], gsem)
            for j in range(KB)]
      ss = []
      for j in range(KB):
        gs[j].wait()
        ss.append(pltpu.async_copy(rows_v.at[j], acc.at[didx_v.at[j]], ssem,
                                   add=True))
      for h in ss:
        h.wait()
      return carry

    lax.fori_loop(0, NBATCH, step, 0)
    plsc.subcore_barrier()
    obase = c * N_PAD + zbase
    for z in range(RPT // ZCH):
      pltpu.sync_copy(acc.at[pl.ds(zbase + z * ZCH, ZCH)], zb_v)
      pltpu.sync_copy(zb_v, out.at[pl.ds(obase + z * ZCH, ZCH)])

  return seg


_sc_cache = {}


def _segsum():
  if "seg" not in _sc_cache:
    _sc_cache["seg"] = _make_segsum()
  return _sc_cache["seg"]


BN = 10000
NB = N_NODES // BN


def _gin_mlp_a(din):
  def body(x_ref, a0_ref, a1_ref, w1_ref, b1_ref, w2_ref, b2_ref,
           h2_ref, st_ref, scr):
    i = pl.program_id(0)
    h = x_ref[...] + a0_ref[...] + a1_ref[...]
    if din == 1:
      h1 = h * w1_ref[...] + b1_ref[...]
    else:
      h1 = jnp.dot(h, w1_ref[...], preferred_element_type=_f32) + b1_ref[...]
    h2 = jnp.dot(h1, w2_ref[...], preferred_element_type=_f32) + b2_ref[...]
    h2_ref[...] = h2
    blk = jnp.stack([jnp.sum(h2, axis=0), jnp.sum(h2 * h2, axis=0)])

    @pl.when(i == 0)
    def _():
      scr[...] = blk

    @pl.when(i > 0)
    def _():
      scr[...] = scr[...] + blk

    st_ref[...] = scr[...]

  return pl.pallas_call(
      body,
      grid=(NB,),
      in_specs=[
          pl.BlockSpec((BN, din), lambda i: (i, 0)),
          pl.BlockSpec((BN, din), lambda i: (i, 0)),
          pl.BlockSpec((BN, din), lambda i: (i, 0)),
          pl.BlockSpec((din, HID), lambda i: (0, 0)),
          pl.BlockSpec((1, HID), lambda i: (0, 0)),
          pl.BlockSpec((HID, HID), lambda i: (0, 0)),
          pl.BlockSpec((1, HID), lambda i: (0, 0)),
      ],
      out_specs=[
          pl.BlockSpec((BN, HID), lambda i: (i, 0)),
          pl.BlockSpec((2, HID), lambda i: (0, 0)),
      ],
      out_shape=[
          jax.ShapeDtypeStruct((N_NODES, HID), _f32),
          jax.ShapeDtypeStruct((2, HID), _f32),
      ],
      scratch_shapes=[pltpu.VMEM((2, HID), _f32)],
  )


_gin_mlp_a1 = _gin_mlp_a(1)
_gin_mlp_a10 = _gin_mlp_a(HID)


def _bn_body(h2_ref, st_ref, g_ref, be_ref, o_ref):
  inv_n = 1.0 / N_NODES
  mu = st_ref[0:1, :] * inv_n
  var = st_ref[1:2, :] * inv_n - mu * mu
  scale = lax.rsqrt(var + 1e-5) * g_ref[...]
  o_ref[...] = (h2_ref[...] - mu) * scale + be_ref[...]


_bn_norm = pl.pallas_call(
    _bn_body,
    grid=(NB,),
    in_specs=[
        pl.BlockSpec((BN, HID), lambda i: (i, 0)),
        pl.BlockSpec((2, HID), lambda i: (0, 0)),
        pl.BlockSpec((1, HID), lambda i: (0, 0)),
        pl.BlockSpec((1, HID), lambda i: (0, 0)),
    ],
    out_specs=pl.BlockSpec((BN, HID), lambda i: (i, 0)),
    out_shape=jax.ShapeDtypeStruct((N_NODES, HID), _f32),
)


def _readout_body(x1_ref, x2_ref, x3_ref, w1_ref, w2_ref, w3_ref, bf_ref,
                  s_ref):
  logits = (
      jnp.dot(x1_ref[...], w1_ref[...], preferred_element_type=_f32)
      + jnp.dot(x2_ref[...], w2_ref[...], preferred_element_type=_f32)
      + jnp.dot(x3_ref[...], w3_ref[...], preferred_element_type=_f32)
      + bf_ref[...])
  m = jnp.max(logits, axis=-1, keepdims=True)
  e = jnp.exp(logits - m)
  s_ref[...] = e / jnp.sum(e, axis=-1, keepdims=True)


_readout = pl.pallas_call(
    _readout_body,
    grid=(NB,),
    in_specs=[
        pl.BlockSpec((BN, HID), lambda i: (i, 0)),
        pl.BlockSpec((BN, HID), lambda i: (i, 0)),
        pl.BlockSpec((BN, HID), lambda i: (i, 0)),
        pl.BlockSpec((HID, OUT_DIM), lambda i: (0, 0)),
        pl.BlockSpec((HID, OUT_DIM), lambda i: (0, 0)),
        pl.BlockSpec((HID, OUT_DIM), lambda i: (0, 0)),
        pl.BlockSpec((1, OUT_DIM), lambda i: (0, 0)),
    ],
    out_specs=pl.BlockSpec((BN, OUT_DIM), lambda i: (i, 0)),
    out_shape=jax.ShapeDtypeStruct((N_NODES, OUT_DIM), _f32),
)

_TWO_M = float(N_EDGES)


BN_M = 4000
NB_M = N_NODES // BN_M


def _mod_body(s_ref, p0_ref, p1_ref, q_ref, scr):
  i = pl.program_id(0)
  sv = s_ref[...]
  agg = p0_ref[...] + p1_ref[...]
  pos_blk = jnp.sum(sv * agg)

  @pl.when(i == 0)
  def _():
    scr[0] = pos_blk
    for j in range(OUT_DIM):
      scr[1 + j] = jnp.sum(agg[:, j:j + 1])

  @pl.when(i > 0)
  def _():
    scr[0] = scr[0] + pos_blk
    for j in range(OUT_DIM):
      scr[1 + j] = scr[1 + j] + jnp.sum(agg[:, j:j + 1])

  neg = (scr[1] * scr[1] + scr[2] * scr[2] + scr[3] * scr[3]
         + scr[4] * scr[4]) / _TWO_M
  q_ref[...] = jnp.full((1, 1), (scr[0] - neg) / _TWO_M)


_modularity = pl.pallas_call(
    _mod_body,
    grid=(NB_M,),
    in_specs=[
        pl.BlockSpec((BN_M, OUT_DIM), lambda i: (i, 0)),
        pl.BlockSpec((BN_M, OUT_DIM), lambda i: (i, 0)),
        pl.BlockSpec((BN_M, OUT_DIM), lambda i: (i, 0)),
    ],
    out_specs=pl.BlockSpec((1, 1), lambda i: (0, 0)),
    out_shape=jax.ShapeDtypeStruct((1, 1), _f32),
    scratch_shapes=[pltpu.SMEM((8,), _f32)],
)


def _gin_layer(xin, part, w1, b1, w2, b2, g, be, din):
  mlp = _gin_mlp_a1 if din == 1 else _gin_mlp_a10
  h2, st = mlp(xin, part[:N_NODES], part[N_PAD:N_PAD + N_NODES],
               w1.reshape(din, HID), b1.reshape(1, HID),
               w2, b2.reshape(1, HID))
  return _bn_norm(h2, st, g.reshape(1, HID), be.reshape(1, HID))


def kernel(x, edge_index, W11, b11, W12, b12, g1, be1, W21, b21, W22, b22,
           g2, be2, W31, b31, W32, b32, g3, be3, Wf, bf):
  src = edge_index[0]
  dst = edge_index[1]
  padv = jnp.full((E_PAD - N_EDGES,), N_NODES, jnp.int32)
  srcp = jnp.concatenate([src, padv]).reshape(-1, CH)
  dstp = jnp.concatenate([dst, padv]).reshape(-1, CH)
  zr = jnp.zeros((ZCH, DP), _f32)

  def table(a):
    return jnp.pad(a, ((0, 1), (0, DP - a.shape[1])))

  p = _segsum()(table(x), srcp, dstp, zr)
  x11 = _gin_layer(x, p[:, :1], W11, b11, W12, b12, g1, be1, 1)

  p = _segsum()(table(x11), srcp, dstp, zr)
  x12 = _gin_layer(x11, p[:, :HID], W21, b21, W22, b22, g2, be2, HID)

  p = _segsum()(table(x12), srcp, dstp, zr)
  x13 = _gin_layer(x12, p[:, :HID], W31, b31, W32, b32, g3, be3, HID)

  s = _readout(x11, x12, x13, Wf[:HID], Wf[HID:2 * HID], Wf[2 * HID:],
               bf.reshape(1, OUT_DIM))

  sp = _segsum()(table(s), srcp, dstp, zr)
  q = _modularity(s, sp[:N_NODES, :OUT_DIM],
                  sp[N_PAD:N_PAD + N_NODES, :OUT_DIM])
  return s, q[0, 0]


# speedup vs baseline: 56.0433x; 1.1154x over previous
import functools

import jax
import jax.numpy as jnp
from jax import lax
from jax.experimental import pallas as pl
from jax.experimental.pallas import tpu as pltpu
from jax.experimental.pallas import tpu_sc as plsc

N_NODES = 100000
N_EDGES = 6400000
HID = 10
OUT_DIM = 4

NC = 2
NS = 16
NW = NC * NS
EPT = 200704
E_PAD = EPT * NW
N_PAD = 100096
RPT = N_PAD // NS
ZCH = 272
DP = 16

_f32 = jnp.float32


def _make_segsum(d, ch, kb):
  cpt = EPT // ch
  nbatch = cpt // kb
  assert nbatch * kb * ch == EPT
  mesh = plsc.VectorSubcoreMesh(core_axis_name="c", subcore_axis_name="s", num_cores=NC, num_subcores=NS)

  @functools.partial(
      pl.kernel,
      mesh=mesh,
      compiler_params=pltpu.CompilerParams(use_tc_tiling_on_sc=False),
      out_type=jax.ShapeDtypeStruct((2 * N_PAD, d), _f32),
      scratch_types=[
          pltpu.VMEM((kb, ch), jnp.int32),
          pltpu.VMEM((kb, ch), jnp.int32),
          pltpu.VMEM((kb, ch, d), _f32),
          pltpu.VMEM((ZCH, d), _f32),
          pltpu.VMEM_SHARED((N_PAD, d), _f32),
          pltpu.SemaphoreType.DMA,
          pltpu.SemaphoreType.DMA,
          pltpu.SemaphoreType.DMA,
      ],
  )
  def seg(table, srcp, dstp, zrows, out, sidx_v, didx_v, rows_v, zb_v, acc,
          isem, gsem, ssem):
    c = lax.axis_index("c")
    s = lax.axis_index("s")
    w = c * NS + s
    pltpu.sync_copy(zrows, zb_v)
    zbase = s * RPT
    for z in range(RPT // ZCH):
      pltpu.sync_copy(zb_v, acc.at[pl.ds(zbase + z * ZCH, ZCH)])
    plsc.subcore_barrier()

    def step(b, carry):
      rbase = w * cpt + b * kb
      i1 = pltpu.async_copy(srcp.at[pl.ds(rbase, kb)], sidx_v, isem)
      i2 = pltpu.async_copy(dstp.at[pl.ds(rbase, kb)], didx_v, isem)
      i1.wait()
      i2.wait()
      gs = [pltpu.async_copy(table.at[sidx_v.at[j]], rows_v.at[j], gsem)
            for j in range(kb)]
      ss = []
      for j in range(kb):
        gs[j].wait()
        ss.append(pltpu.async_copy(rows_v.at[j], acc.at[didx_v.at[j]], ssem,
                                   add=True))
      for h in ss:
        h.wait()
      return carry

    lax.fori_loop(0, nbatch, step, 0)
    plsc.subcore_barrier()
    obase = c * N_PAD + zbase
    for z in range(RPT // ZCH):
      pltpu.sync_copy(acc.at[pl.ds(zbase + z * ZCH, ZCH)], zb_v)
      pltpu.sync_copy(zb_v, out.at[pl.ds(obase + z * ZCH, ZCH)])

  return seg


_sc_cache = {}


def _segsum(d, ch, kb):
  key = ("seg", d, ch, kb)
  if key not in _sc_cache:
    _sc_cache[key] = _make_segsum(d, ch, kb)
  return _sc_cache[key]


BN = 10000
NB = N_NODES // BN


def _gin_mlp_a(din):
  def body(x_ref, a0_ref, a1_ref, w1_ref, b1_ref, w2_ref, b2_ref,
           h2_ref, st_ref, scr):
    i = pl.program_id(0)
    h = x_ref[...] + a0_ref[...] + a1_ref[...]
    if din == 1:
      h1 = h * w1_ref[...] + b1_ref[...]
    else:
      h1 = jnp.dot(h, w1_ref[...], preferred_element_type=_f32) + b1_ref[...]
    h2 = jnp.dot(h1, w2_ref[...], preferred_element_type=_f32) + b2_ref[...]
    h2_ref[...] = h2
    blk = jnp.stack([jnp.sum(h2, axis=0), jnp.sum(h2 * h2, axis=0)])

    @pl.when(i == 0)
    def _():
      scr[...] = blk

    @pl.when(i > 0)
    def _():
      scr[...] = scr[...] + blk

    st_ref[...] = scr[...]

  return pl.pallas_call(
      body,
      grid=(NB,),
      in_specs=[
          pl.BlockSpec((BN, din), lambda i: (i, 0)),
          pl.BlockSpec((BN, din), lambda i: (i, 0)),
          pl.BlockSpec((BN, din), lambda i: (i, 0)),
          pl.BlockSpec((din, HID), lambda i: (0, 0)),
          pl.BlockSpec((1, HID), lambda i: (0, 0)),
          pl.BlockSpec((HID, HID), lambda i: (0, 0)),
          pl.BlockSpec((1, HID), lambda i: (0, 0)),
      ],
      out_specs=[
          pl.BlockSpec((BN, HID), lambda i: (i, 0)),
          pl.BlockSpec((2, HID), lambda i: (0, 0)),
      ],
      out_shape=[
          jax.ShapeDtypeStruct((N_NODES, HID), _f32),
          jax.ShapeDtypeStruct((2, HID), _f32),
      ],
      scratch_shapes=[pltpu.VMEM((2, HID), _f32)],
  )


_gin_mlp_a1 = _gin_mlp_a(1)
_gin_mlp_a10 = _gin_mlp_a(HID)


def _bn_body(h2_ref, st_ref, g_ref, be_ref, o_ref):
  inv_n = 1.0 / N_NODES
  mu = st_ref[0:1, :] * inv_n
  var = st_ref[1:2, :] * inv_n - mu * mu
  scale = lax.rsqrt(var + 1e-5) * g_ref[...]
  o_ref[...] = (h2_ref[...] - mu) * scale + be_ref[...]


_bn_norm = pl.pallas_call(
    _bn_body,
    grid=(NB,),
    in_specs=[
        pl.BlockSpec((BN, HID), lambda i: (i, 0)),
        pl.BlockSpec((2, HID), lambda i: (0, 0)),
        pl.BlockSpec((1, HID), lambda i: (0, 0)),
        pl.BlockSpec((1, HID), lambda i: (0, 0)),
    ],
    out_specs=pl.BlockSpec((BN, HID), lambda i: (i, 0)),
    out_shape=jax.ShapeDtypeStruct((N_NODES, HID), _f32),
)


def _readout_body(x1_ref, x2_ref, x3_ref, w1_ref, w2_ref, w3_ref, bf_ref,
                  s_ref):
  logits = (
      jnp.dot(x1_ref[...], w1_ref[...], preferred_element_type=_f32)
      + jnp.dot(x2_ref[...], w2_ref[...], preferred_element_type=_f32)
      + jnp.dot(x3_ref[...], w3_ref[...], preferred_element_type=_f32)
      + bf_ref[...])
  m = jnp.max(logits, axis=-1, keepdims=True)
  e = jnp.exp(logits - m)
  s_ref[...] = e / jnp.sum(e, axis=-1, keepdims=True)


_readout = pl.pallas_call(
    _readout_body,
    grid=(NB,),
    in_specs=[
        pl.BlockSpec((BN, HID), lambda i: (i, 0)),
        pl.BlockSpec((BN, HID), lambda i: (i, 0)),
        pl.BlockSpec((BN, HID), lambda i: (i, 0)),
        pl.BlockSpec((HID, OUT_DIM), lambda i: (0, 0)),
        pl.BlockSpec((HID, OUT_DIM), lambda i: (0, 0)),
        pl.BlockSpec((HID, OUT_DIM), lambda i: (0, 0)),
        pl.BlockSpec((1, OUT_DIM), lambda i: (0, 0)),
    ],
    out_specs=pl.BlockSpec((BN, OUT_DIM), lambda i: (i, 0)),
    out_shape=jax.ShapeDtypeStruct((N_NODES, OUT_DIM), _f32),
)

_TWO_M = float(N_EDGES)


BN_M = 4000
NB_M = N_NODES // BN_M


def _mod_body(s_ref, p0_ref, p1_ref, q_ref, scr):
  i = pl.program_id(0)
  sv = s_ref[...]
  agg = p0_ref[...] + p1_ref[...]
  pos_blk = jnp.sum(sv * agg)

  @pl.when(i == 0)
  def _():
    scr[0] = pos_blk
    for j in range(OUT_DIM):
      scr[1 + j] = jnp.sum(agg[:, j:j + 1])

  @pl.when(i > 0)
  def _():
    scr[0] = scr[0] + pos_blk
    for j in range(OUT_DIM):
      scr[1 + j] = scr[1 + j] + jnp.sum(agg[:, j:j + 1])

  neg = (scr[1] * scr[1] + scr[2] * scr[2] + scr[3] * scr[3]
         + scr[4] * scr[4]) / _TWO_M
  q_ref[...] = jnp.full((1, 1), (scr[0] - neg) / _TWO_M)


_modularity = pl.pallas_call(
    _mod_body,
    grid=(NB_M,),
    in_specs=[
        pl.BlockSpec((BN_M, OUT_DIM), lambda i: (i, 0)),
        pl.BlockSpec((BN_M, OUT_DIM), lambda i: (i, 0)),
        pl.BlockSpec((BN_M, OUT_DIM), lambda i: (i, 0)),
    ],
    out_specs=pl.BlockSpec((1, 1), lambda i: (0, 0)),
    out_shape=jax.ShapeDtypeStruct((1, 1), _f32),
    scratch_shapes=[pltpu.SMEM((8,), _f32)],
)


def _gin_layer(xin, part, w1, b1, w2, b2, g, be, din):
  mlp = _gin_mlp_a1 if din == 1 else _gin_mlp_a10
  h2, st = mlp(xin, part[:N_NODES], part[N_PAD:N_PAD + N_NODES],
               w1.reshape(din, HID), b1.reshape(1, HID),
               w2, b2.reshape(1, HID))
  return _bn_norm(h2, st, g.reshape(1, HID), be.reshape(1, HID))


def kernel(x, edge_index, W11, b11, W12, b12, g1, be1, W21, b21, W22, b22,
           g2, be2, W31, b31, W32, b32, g3, be3, Wf, bf):
  src = edge_index[0]
  dst = edge_index[1]
  padv = jnp.full((E_PAD - N_EDGES,), N_NODES, jnp.int32)
  srcf = jnp.concatenate([src, padv])
  dstf = jnp.concatenate([dst, padv])
  srcp256 = srcf.reshape(-1, 256)
  dstp256 = dstf.reshape(-1, 256)
  srcp512 = srcf.reshape(-1, 512)
  dstp512 = dstf.reshape(-1, 512)
  zr16 = jnp.zeros((ZCH, 16), _f32)
  zr8 = jnp.zeros((ZCH, 8), _f32)

  def table(a, d):
    return jnp.pad(a, ((0, 1), (0, d - a.shape[1])))

  seg16 = _segsum(16, 256, 4)
  seg8 = _segsum(8, 512, 8)

  p = seg8(table(x, 8), srcp512, dstp512, zr8)
  x11 = _gin_layer(x, p[:, :1], W11, b11, W12, b12, g1, be1, 1)

  p = seg16(table(x11, 16), srcp256, dstp256, zr16)
  x12 = _gin_layer(x11, p[:, :HID], W21, b21, W22, b22, g2, be2, HID)

  p = seg16(table(x12, 16), srcp256, dstp256, zr16)
  x13 = _gin_layer(x12, p[:, :HID], W31, b31, W32, b32, g3, be3, HID)

  s = _readout(x11, x12, x13, Wf[:HID], Wf[HID:2 * HID], Wf[2 * HID:],
               bf.reshape(1, OUT_DIM))

  sp = seg8(table(s, 8), srcp512, dstp512, zr8)
  q = _modularity(s, sp[:N_NODES, :OUT_DIM],
                  sp[N_PAD:N_PAD + N_NODES, :OUT_DIM])
  return s, q[0, 0]
